# baseline (speedup 1.0000x reference)
import jax
import jax.numpy as jnp
from jax import lax
from jax.experimental import pallas as pl
from jax.experimental.pallas import tpu as pltpu, tpu_sc as plsc

NCLS = 1000
K = 20
N = 4096
NW = 32
STRIP = 8
NSTRIPS = (K * NCLS) // STRIP
BASE = NSTRIPS // NW
EXTRA = NSTRIPS - BASE * NW
NCHUNK = N // 16


def _sc_body(xt_hbm, out_hbm, xcol, buf):
    cid = lax.axis_index("c")
    sid = lax.axis_index("s")
    wid = sid * 2 + cid
    ns = BASE + jnp.where(wid < EXTRA, 1, 0)
    s0 = wid * BASE + jnp.minimum(wid, EXTRA)

    def strip_body(si, prev_k):
        s = s0 + si
        k = s // (NCLS // STRIP)
        c0 = (s % (NCLS // STRIP)) * STRIP

        @pl.when(k != prev_k)
        def _():
            pltpu.sync_copy(xt_hbm.at[pl.ds(k * N, N)], xcol)

        def chunk(t, _):
            vals = xcol[pl.ds(t * 16, 16)]
            r = vals - c0
            for c in range(STRIP):
                buf[c, pl.ds(t * 16, 16)] = (r == c).astype(jnp.int32)
            return 0

        lax.fori_loop(0, NCHUNK, chunk, 0)
        pltpu.sync_copy(buf, out_hbm.at[pl.ds(s * STRIP, STRIP), :])
        return k

    lax.fori_loop(0, ns, strip_body, jnp.int32(-1))


def sc_kernel(x):
    out_dtype = jax.dtypes.canonicalize_dtype(jnp.int64)
    n, k = x.shape
    xt = x.astype(jnp.int32).T.reshape(k * n)
    mesh = plsc.VectorSubcoreMesh(core_axis_name="c", subcore_axis_name="s")
    out = pl.kernel(
        _sc_body,
        out_type=jax.ShapeDtypeStruct((k * NCLS, n), out_dtype),
        mesh=mesh,
        scratch_types=[
            pltpu.VMEM((n,), jnp.int32),
            pltpu.VMEM((STRIP, n), jnp.int32),
        ],
        compiler_params=pltpu.CompilerParams(needs_layout_passes=False),
    )(xt)
    return out.reshape(k, NCLS, n).transpose(2, 0, 1)


kernel = sc_kernel


# speedup vs baseline: 1.5557x; 1.5557x over previous
import jax
import jax.numpy as jnp
from jax import lax
from jax.experimental import pallas as pl
from jax.experimental.pallas import tpu as pltpu, tpu_sc as plsc

NCLS = 1000
K = 20
N = 4096
NW = 32
STRIP = 8
SPK = NCLS // STRIP
NSTRIPS = K * SPK
BASE = NSTRIPS // NW
EXTRA = NSTRIPS - BASE * NW
NCHUNK = N // 16


def _sc_body(xt_hbm, out_hbm, xcol, buf0, buf1, sem0, sem1):
    cid = lax.axis_index("c")
    sid = lax.axis_index("s")
    wid = sid * 2 + cid
    ns = BASE + jnp.where(wid < EXTRA, 1, 0)
    s0 = wid * BASE + jnp.minimum(wid, EXTRA)

    k_lo = s0 // SPK
    k_hi = (s0 + ns - 1) // SPK
    pltpu.sync_copy(xt_hbm.at[pl.ds(k_lo * N, N)], xcol.at[pl.ds(0, N)])

    @pl.when(k_hi > k_lo)
    def _():
        pltpu.sync_copy(xt_hbm.at[pl.ds(k_hi * N, N)], xcol.at[pl.ds(N, N)])

    def fill(s, buf):
        kk = s // SPK - k_lo
        c0 = (s % SPK) * STRIP

        def chunk(t, _):
            vals = xcol[pl.ds(kk * N + t * 16, 16)]
            r = vals - c0
            for c in range(STRIP):
                buf[c, pl.ds(t * 16, 16)] = (r == c).astype(jnp.int32)
            return 0

        lax.fori_loop(0, NCHUNK, chunk, 0)

    def fire(s, buf, sem):
        pltpu.async_copy(buf, out_hbm.at[pl.ds(s * STRIP, STRIP), :], sem)

    def drain(buf, sem):
        pltpu.make_async_copy(buf, out_hbm.at[pl.ds(0, STRIP), :], sem).wait()

    npairs = (ns + 1) // 2

    def pair(g, _):
        sa = s0 + 2 * g
        sb = sa + 1

        @pl.when(g > 0)
        def _():
            drain(buf0, sem0)

        fill(sa, buf0)
        fire(sa, buf0, sem0)

        @pl.when(g > 0)
        def _():
            drain(buf1, sem1)

        @pl.when(sb < s0 + ns)
        def _():
            fill(sb, buf1)
            fire(sb, buf1, sem1)

        return 0

    lax.fori_loop(0, npairs, pair, 0)
    drain(buf0, sem0)

    @pl.when(ns % 2 == 0)
    def _():
        drain(buf1, sem1)


def sc_kernel(x):
    out_dtype = jax.dtypes.canonicalize_dtype(jnp.int64)
    n, k = x.shape
    xt = x.astype(jnp.int32).T.reshape(k * n)
    mesh = plsc.VectorSubcoreMesh(core_axis_name="c", subcore_axis_name="s")
    out = pl.kernel(
        _sc_body,
        out_type=jax.ShapeDtypeStruct((k * NCLS, n), out_dtype),
        mesh=mesh,
        scratch_types=[
            pltpu.VMEM((2 * n,), jnp.int32),
            pltpu.VMEM((STRIP, n), jnp.int32),
            pltpu.VMEM((STRIP, n), jnp.int32),
            pltpu.SemaphoreType.DMA,
            pltpu.SemaphoreType.DMA,
        ],
        compiler_params=pltpu.CompilerParams(needs_layout_passes=False),
    )(xt)
    return out.reshape(k, NCLS, n).transpose(2, 0, 1)


kernel = sc_kernel


# speedup vs baseline: 3.0170x; 1.9393x over previous
import jax
import jax.numpy as jnp
from jax import lax
from jax.experimental import pallas as pl

NCLS = 1000
BJ = 2048


def _onehot_block(x_ref, o_ref):
    i = pl.program_id(0)
    j = pl.program_id(1)
    xrow = x_ref[pl.ds(i, 1), pl.ds(j * BJ, BJ)]
    cls = lax.broadcasted_iota(jnp.int32, (NCLS, BJ), 0)
    o_ref[0] = (xrow == cls).astype(o_ref.dtype)


def kernel(x):
    out_dtype = jax.dtypes.canonicalize_dtype(jnp.int64)
    n, k = x.shape
    nj = n // BJ
    xt = x.astype(jnp.int32).T
    out = pl.pallas_call(
        _onehot_block,
        grid=(k, nj),
        in_specs=[pl.BlockSpec((k, n), lambda i, j: (0, 0))],
        out_specs=pl.BlockSpec((1, NCLS, BJ), lambda i, j: (i, 0, j)),
        out_shape=jax.ShapeDtypeStruct((k, NCLS, n), out_dtype),
    )(xt)
    return out.transpose(2, 0, 1)
